# baseline (speedup 1.0000x reference)
import jax
import jax.numpy as jnp
from jax import lax
from jax.experimental import pallas as pl
from jax.experimental.pallas import tpu as pltpu
from jax.experimental.pallas import tpu_sc as plsc

N = 10000
E = 160000
D = 256
H = 128
NC = 2
NS = 16
B = 128
NB = 80
NBUF = 2
NI = 4
EPT = NB * B
EPAD = NS * EPT
NPAD = 10112
RPT = NPAD // NS


def _sc_aggregate_body(emb2_hbm, idx_hbm, vals_hbm, out_hbm,
                       islot, vslot, bufs, acc,
                       i0, i1, i2, i3, g0, g1, s0, s1):
    isems = (i0, i1, i2, i3)
    gsems = (g0, g1)
    ssems = (s0, s1)
    c = lax.axis_index("c")
    s = lax.axis_index("s")

    def idx_start(j, r):
        pltpu.async_copy(idx_hbm.at[c, s, j], islot.at[r], isems[r])
        pltpu.async_copy(vals_hbm.at[s, j], vslot.at[r], isems[r])

    def idx_wait(j, r):
        pltpu.make_async_copy(idx_hbm.at[c, s, j], islot.at[r],
                              isems[r]).wait()
        pltpu.make_async_copy(vals_hbm.at[s, j], vslot.at[r],
                              isems[r]).wait()

    def gather_start(r, b):
        pltpu.async_copy(emb2_hbm.at[islot.at[r, 0]], bufs.at[b], gsems[b])

    def gather_wait(r, b):
        pltpu.make_async_copy(emb2_hbm.at[islot.at[r, 0]], bufs.at[b],
                              gsems[b]).wait()

    def scatter_start(r, b):
        pltpu.async_copy(bufs.at[b], acc.at[islot.at[r, 1]], ssems[b],
                         add=True)

    def scatter_wait(r, b):
        pltpu.make_async_copy(bufs.at[b], acc.at[islot.at[r, 1]],
                              ssems[b]).wait()

    for u in range(3):
        idx_start(u, u)

    zv = jnp.zeros((16,), jnp.float32)
    zbuf = bufs.at[0]

    def zrow(k, _):
        for q in range(H // 16):
            zbuf[k, pl.ds(q * 16, 16)] = zv
        return 0

    lax.fori_loop(0, B, zrow, 0)
    for q in range(4):
        pltpu.sync_copy(zbuf, acc.at[pl.ds(s * RPT + q * B, B)])
    pltpu.sync_copy(zbuf.at[pl.ds(0, RPT - 4 * B)],
                    acc.at[pl.ds(s * RPT + 4 * B, RPT - 4 * B)])

    idx_wait(0, 0)
    gather_start(0, 0)
    plsc.subcore_barrier()

    def scale(r, b):
        buf = bufs.at[b]

        def group(g, _):
            vrow = vslot[r, 0, pl.ds(g * 16, 16)]
            for l in range(16):
                v = vrow[l]
                k = g * 16 + l
                for q in range(H // 16):
                    sl = pl.ds(q * 16, 16)
                    buf[k, sl] = buf[k, sl] * v
            return 0

        lax.fori_loop(0, B // 16, group, 0)

    def round_(jj, _):
        for u in range(NI):
            j = jj * NI + u
            b = u % NBUF

            @pl.when(j >= 1)
            def _():
                scatter_wait((u + 3) % NI, (u + 1) % NBUF)

            @pl.when(j + 1 < NB)
            def _():
                idx_wait(j + 1, (u + 1) % NI)
                gather_start((u + 1) % NI, (u + 1) % NBUF)

            @pl.when(j + 3 < NB)
            def _():
                idx_start(j + 3, (u + 3) % NI)

            gather_wait(u, b)
            scale(u, b)
            scatter_start(u, b)
        return 0

    lax.fori_loop(0, NB // NI, round_, 0)

    scatter_wait((NB - 1) % NI, (NB - 1) % NBUF)
    plsc.subcore_barrier()

    pltpu.sync_copy(acc.at[pl.ds(s * RPT, RPT)],
                    out_hbm.at[c, pl.ds(s * RPT, RPT)])


@jax.jit
def _sc_aggregate(emb2, idx5, vals4):
    mesh = plsc.VectorSubcoreMesh(core_axis_name="c", subcore_axis_name="s")
    return pl.kernel(
        _sc_aggregate_body,
        out_type=jax.ShapeDtypeStruct((NC, NPAD, H), jnp.float32),
        mesh=mesh,
        scratch_types=[
            pltpu.VMEM((NI, 2, B), jnp.int32),
            pltpu.VMEM((NI, 1, B), jnp.float32),
            pltpu.VMEM((NBUF, B, H), jnp.float32),
            pltpu.VMEM_SHARED((NPAD, H), jnp.float32),
        ] + [pltpu.SemaphoreType.DMA] * (NI + 2 * NBUF),
    )(emb2, idx5, vals4)


def _tc_mm1_body(emb_r, w1_r, out_r):
    out_r[...] = jnp.dot(emb_r[...], w1_r[...],
                         preferred_element_type=jnp.float32)


@jax.jit
def _tc_mm1(emb, w1t):
    blk = 1000
    return pl.pallas_call(
        _tc_mm1_body,
        grid=(N // blk,),
        in_specs=[
            pl.BlockSpec((blk, D), lambda i: (i, 0)),
            pl.BlockSpec((D, D), lambda i: (0, 0)),
        ],
        out_specs=pl.BlockSpec((blk, D), lambda i: (i, 0)),
        out_shape=jax.ShapeDtypeStruct((N, D), jnp.float32),
    )(emb, w1t)


def _tc_mm2_body(a_r, n0_r, n1_r, w2a_r, w2b_r, out_r):
    x = a_r[...]
    x += jnp.dot(n0_r[0], w2a_r[...], preferred_element_type=jnp.float32)
    x += jnp.dot(n1_r[0], w2b_r[...], preferred_element_type=jnp.float32)
    out_r[...] = jnp.where(x >= 0, x, 0.2 * x)


@jax.jit
def _tc_mm2(a, nb, w2ta, w2tb):
    blk = 1000
    return pl.pallas_call(
        _tc_mm2_body,
        grid=(N // blk,),
        in_specs=[
            pl.BlockSpec((blk, D), lambda i: (i, 0)),
            pl.BlockSpec((1, blk, H), lambda i: (0, i, 0)),
            pl.BlockSpec((1, blk, H), lambda i: (1, i, 0)),
            pl.BlockSpec((H, D), lambda i: (0, 0)),
            pl.BlockSpec((H, D), lambda i: (0, 0)),
        ],
        out_specs=pl.BlockSpec((blk, D), lambda i: (i, 0)),
        out_shape=jax.ShapeDtypeStruct((N, D), jnp.float32),
    )(a, nb, nb, w2ta, w2tb)


def kernel(emb, adj_indices, adj_values, W1, W2):
    rows = adj_indices[0]
    cols = adj_indices[1]
    pad = EPAD - E
    rows_p = jnp.concatenate([rows, jnp.zeros((pad,), jnp.int32)])
    cols_p = jnp.concatenate([cols, jnp.zeros((pad,), jnp.int32)])
    vals_p = jnp.concatenate([adj_values, jnp.zeros((pad,), jnp.float32)])

    emb2 = emb.reshape(N * NC, H)

    colsx = cols_p * 2
    idx5 = jnp.stack([
        jnp.stack([colsx, rows_p]),
        jnp.stack([colsx + 1, rows_p]),
    ]).reshape(NC, 2, NS, NB, B).transpose(0, 2, 3, 1, 4)
    vals4 = vals_p.reshape(NS, NB, 1, B)

    nb = _sc_aggregate(emb2, idx5, vals4)
    a = _tc_mm1(emb, W1.T)
    return _tc_mm2(a, nb, W2[:, :H].T, W2[:, H:].T)


# speedup vs baseline: 1.4486x; 1.4486x over previous
import jax
import jax.numpy as jnp
from jax import lax
from jax.experimental import pallas as pl
from jax.experimental.pallas import tpu as pltpu
from jax.experimental.pallas import tpu_sc as plsc

N = 10000
E = 160000
D = 256
H = 128
NC = 2
NS = 16
B = 112
NB = 90
NBUF = 3
NI = 6
EPT = NB * B
EPAD = NS * EPT
NPAD = 10112
RPT = NPAD // NS


def _sc_aggregate_body(emb2_hbm, cols_hbm, rows_hbm, vals_hbm, out_hbm,
                       cslot, rslot, vslot, bufs, acc,
                       i0, i1, i2, i3, i4, i5, g0, g1, g2, s0, s1, s2):
    isems = (i0, i1, i2, i3, i4, i5)
    gsems = (g0, g1, g2)
    ssems = (s0, s1, s2)
    c = lax.axis_index("c")
    s = lax.axis_index("s")

    def idx_start(j, r):
        pltpu.async_copy(cols_hbm.at[c, s, j], cslot.at[r], isems[r])
        pltpu.async_copy(rows_hbm.at[s, j], rslot.at[r], isems[r])
        pltpu.async_copy(vals_hbm.at[s, j], vslot.at[r], isems[r])

    def idx_wait(j, r):
        pltpu.make_async_copy(cols_hbm.at[c, s, j], cslot.at[r],
                              isems[r]).wait()
        pltpu.make_async_copy(rows_hbm.at[s, j], rslot.at[r],
                              isems[r]).wait()
        pltpu.make_async_copy(vals_hbm.at[s, j], vslot.at[r],
                              isems[r]).wait()

    def gather_start(j, r, b):
        pltpu.async_copy(emb2_hbm.at[cslot.at[r, 0]], bufs.at[b], gsems[b])

    def gather_wait(r, b):
        pltpu.make_async_copy(emb2_hbm.at[cslot.at[r, 0]], bufs.at[b],
                              gsems[b]).wait()

    def scatter_start(r, b):
        pltpu.async_copy(bufs.at[b], acc.at[rslot.at[r, 0]], ssems[b],
                         add=True)

    def scatter_wait(r, b):
        pltpu.make_async_copy(bufs.at[b], acc.at[rslot.at[r, 0]],
                              ssems[b]).wait()

    for u in range(4):
        idx_start(u, u)

    zv = jnp.zeros((16,), jnp.float32)
    zbuf = bufs.at[0]

    def zrow(k, _):
        for q in range(H // 16):
            zbuf[k, pl.ds(q * 16, 16)] = zv
        return 0

    lax.fori_loop(0, B, zrow, 0)
    for q in range(5):
        pltpu.sync_copy(zbuf, acc.at[pl.ds(s * RPT + q * B, B)])
    pltpu.sync_copy(zbuf.at[pl.ds(0, RPT - 5 * B)],
                    acc.at[pl.ds(s * RPT + 5 * B, RPT - 5 * B)])
    plsc.subcore_barrier()

    idx_wait(0, 0)
    gather_start(0, 0, 0)
    idx_wait(1, 1)
    gather_start(1, 1, 1)

    def scale(b, r):
        buf = bufs.at[b]

        def group(g, _):
            vrow = vslot[r, 0, pl.ds(g * 16, 16)]
            for l in range(16):
                v = vrow[l]
                k = g * 16 + l
                for q in range(H // 16):
                    sl = pl.ds(q * 16, 16)
                    buf[k, sl] = buf[k, sl] * v
            return 0

        lax.fori_loop(0, B // 16, group, 0)

    def round_(jj, _):
        for u in range(NI):
            j = jj * NI + u
            b = u % NBUF
            r = u

            @pl.when(j >= 1)
            def _():
                scatter_wait((u + 5) % NI, (u + 2) % NBUF)

            @pl.when(j + 4 < NB)
            def _():
                idx_start(j + 4, (u + 4) % NI)

            @pl.when(j + 2 < NB)
            def _():
                idx_wait(j + 2, (u + 2) % NI)
                gather_start(j + 2, (u + 2) % NI, (u + 2) % NBUF)

            gather_wait(r, b)
            scale(b, r)
            scatter_start(r, b)
        return 0

    lax.fori_loop(0, NB // NI, round_, 0)

    scatter_wait((NB - 1) % NI, (NB - 1) % NBUF)
    plsc.subcore_barrier()

    pltpu.sync_copy(acc.at[pl.ds(s * RPT, RPT)],
                    out_hbm.at[c, pl.ds(s * RPT, RPT)])


@jax.jit
def _sc_aggregate(emb2, cols5, rows4, vals4):
    mesh = plsc.VectorSubcoreMesh(core_axis_name="c", subcore_axis_name="s")
    return pl.kernel(
        _sc_aggregate_body,
        out_type=jax.ShapeDtypeStruct((NC, NPAD, H), jnp.float32),
        mesh=mesh,
        scratch_types=[
            pltpu.VMEM((NI, 1, B), jnp.int32),
            pltpu.VMEM((NI, 1, B), jnp.int32),
            pltpu.VMEM((NI, 1, B), jnp.float32),
            pltpu.VMEM((NBUF, B, H), jnp.float32),
            pltpu.VMEM_SHARED((NPAD, H), jnp.float32),
        ] + [pltpu.SemaphoreType.DMA] * (NI + 2 * NBUF),
    )(emb2, cols5, rows4, vals4)


def _tc_mm1_body(emb_r, w1_r, out_r):
    out_r[...] = jnp.dot(emb_r[...], w1_r[...],
                         preferred_element_type=jnp.float32)


@jax.jit
def _tc_mm1(emb, w1t):
    blk = 1000
    return pl.pallas_call(
        _tc_mm1_body,
        grid=(N // blk,),
        in_specs=[
            pl.BlockSpec((blk, D), lambda i: (i, 0)),
            pl.BlockSpec((D, D), lambda i: (0, 0)),
        ],
        out_specs=pl.BlockSpec((blk, D), lambda i: (i, 0)),
        out_shape=jax.ShapeDtypeStruct((N, D), jnp.float32),
    )(emb, w1t)


def _tc_mm2_body(a_r, n0_r, n1_r, w2a_r, w2b_r, out_r):
    x = a_r[...]
    x += jnp.dot(n0_r[0], w2a_r[...], preferred_element_type=jnp.float32)
    x += jnp.dot(n1_r[0], w2b_r[...], preferred_element_type=jnp.float32)
    out_r[...] = jnp.where(x >= 0, x, 0.2 * x)


@jax.jit
def _tc_mm2(a, nb, w2ta, w2tb):
    blk = 1000
    return pl.pallas_call(
        _tc_mm2_body,
        grid=(N // blk,),
        in_specs=[
            pl.BlockSpec((blk, D), lambda i: (i, 0)),
            pl.BlockSpec((1, blk, H), lambda i: (0, i, 0)),
            pl.BlockSpec((1, blk, H), lambda i: (1, i, 0)),
            pl.BlockSpec((H, D), lambda i: (0, 0)),
            pl.BlockSpec((H, D), lambda i: (0, 0)),
        ],
        out_specs=pl.BlockSpec((blk, D), lambda i: (i, 0)),
        out_shape=jax.ShapeDtypeStruct((N, D), jnp.float32),
    )(a, nb, nb, w2ta, w2tb)


def kernel(emb, adj_indices, adj_values, W1, W2):
    rows = adj_indices[0]
    cols = adj_indices[1]
    pad = EPAD - E
    rows_p = jnp.concatenate([rows, jnp.zeros((pad,), jnp.int32)])
    cols_p = jnp.concatenate([cols, jnp.zeros((pad,), jnp.int32)])
    vals_p = jnp.concatenate([adj_values, jnp.zeros((pad,), jnp.float32)])

    emb2 = emb.reshape(N * NC, H)
    colsx = cols_p * 2
    cols5 = jnp.stack([colsx, colsx + 1]).reshape(NC, NS, NB, 1, B)
    rows4 = rows_p.reshape(NS, NB, 1, B)
    vals4 = vals_p.reshape(NS, NB, 1, B)

    nb = _sc_aggregate(emb2, cols5, rows4, vals4)
    a = _tc_mm1(emb, W1.T)
    return _tc_mm2(a, nb, W2[:, :H].T, W2[:, H:].T)


# speedup vs baseline: 1.5438x; 1.0657x over previous
import jax
import jax.numpy as jnp
from jax import lax
from jax.experimental import pallas as pl
from jax.experimental.pallas import tpu as pltpu
from jax.experimental.pallas import tpu_sc as plsc

N = 10000
E = 160000
D = 256
H = 128
NC = 2
NS = 16
B = 112
NB = 90
NBUF = 3
NI = 6
EPT = NB * B
EPAD = NS * EPT
NPAD = 10112
RPT = NPAD // NS


BP = 128
CHW = NI * BP


def _sc_aggregate_body(emb2_hbm, cols_hbm, rows_hbm, vals_hbm, out_hbm,
                       cring, rslot, vring, bufs, acc,
                       i0, i1, i2, i3, i4, i5, g0, g1, g2, s0, s1, s2, ks):
    isems = (i0, i1, i2, i3, i4, i5)
    gsems = (g0, g1, g2)
    ssems = (s0, s1, s2)
    c = lax.axis_index("c")
    s = lax.axis_index("s")
    base_c = (c * NS + s) * (NB * BP)
    base_v = s * (NB * BP)

    def chunk_start(q, d):
        pltpu.async_copy(cols_hbm.at[pl.ds(base_c + q * CHW, CHW)],
                         cring.at[d, 0], ks)
        pltpu.async_copy(vals_hbm.at[pl.ds(base_v + q * CHW, CHW)],
                         vring.at[d, 0], ks)

    def chunk_wait(q, d):
        pltpu.make_async_copy(cols_hbm.at[pl.ds(base_c + q * CHW, CHW)],
                              cring.at[d, 0], ks).wait()
        pltpu.make_async_copy(vals_hbm.at[pl.ds(base_v + q * CHW, CHW)],
                              vring.at[d, 0], ks).wait()

    def idx_start(j, r):
        pltpu.async_copy(rows_hbm.at[s, j], rslot.at[r], isems[r])

    def idx_wait(j, r):
        pltpu.make_async_copy(rows_hbm.at[s, j], rslot.at[r],
                              isems[r]).wait()

    def gather_start(m, d, b):
        pltpu.async_copy(emb2_hbm.at[cring.at[d, 0, pl.ds(m * BP, B)]],
                         bufs.at[b], gsems[b])

    def gather_wait(m, d, b):
        pltpu.make_async_copy(emb2_hbm.at[cring.at[d, 0, pl.ds(m * BP, B)]],
                              bufs.at[b], gsems[b]).wait()

    def scatter_start(r, b):
        pltpu.async_copy(bufs.at[b], acc.at[rslot.at[r, 0]], ssems[b],
                         add=True)

    def scatter_wait(r, b):
        pltpu.make_async_copy(bufs.at[b], acc.at[rslot.at[r, 0]],
                              ssems[b]).wait()

    chunk_start(0, 0)
    for u in range(4):
        idx_start(u, u)

    zv = jnp.zeros((16,), jnp.float32)
    zbuf = bufs.at[0]

    def zrow(k, _):
        for q in range(H // 16):
            zbuf[k, pl.ds(q * 16, 16)] = zv
        return 0

    lax.fori_loop(0, B, zrow, 0)
    for q in range(5):
        pltpu.sync_copy(zbuf, acc.at[pl.ds(s * RPT + q * B, B)])
    pltpu.sync_copy(zbuf.at[pl.ds(0, RPT - 5 * B)],
                    acc.at[pl.ds(s * RPT + 5 * B, RPT - 5 * B)])
    plsc.subcore_barrier()

    idx_wait(0, 0)
    idx_wait(1, 1)
    chunk_wait(0, 0)
    gather_start(0, 0, 0)
    gather_start(1, 0, 1)

    def scale(b, u, dv):
        buf = bufs.at[b]

        def group(g, _):
            vrow = vring[dv, 0, pl.ds(u * BP + g * 16, 16)]
            for l in range(16):
                v = vrow[l]
                k = g * 16 + l
                for q in range(H // 16):
                    sl = pl.ds(q * 16, 16)
                    buf[k, sl] = buf[k, sl] * v
            return 0

        lax.fori_loop(0, B // 16, group, 0)

    def round_(jj, _):
        d0 = lax.rem(jj, 2)
        d1 = lax.rem(jj + 1, 2)
        for u in range(NI):
            j = jj * NI + u
            b = u % NBUF

            @pl.when(j >= 1)
            def _():
                scatter_wait((u + 5) % NI, (u + 2) % NBUF)

            @pl.when(j + 4 < NB)
            def _():
                idx_start(j + 4, (u + 4) % NI)

            if u == 2:
                @pl.when(j + 4 < NB)
                def _():
                    chunk_start(jj + 1, d1)

            @pl.when(j + 2 < NB)
            def _():
                idx_wait(j + 2, (u + 2) % NI)
                if u == 4:
                    chunk_wait(jj + 1, d1)
                if u < 4:
                    gather_start(u + 2, d0, (u + 2) % NBUF)
                else:
                    gather_start(u - 4, d1, (u + 2) % NBUF)

            gather_wait(u, d0, b)
            scale(b, u, d0)
            scatter_start(u, b)
        return 0

    lax.fori_loop(0, NB // NI, round_, 0)

    scatter_wait((NB - 1) % NI, (NB - 1) % NBUF)
    plsc.subcore_barrier()

    pltpu.sync_copy(acc.at[pl.ds(s * RPT, RPT)],
                    out_hbm.at[c, pl.ds(s * RPT, RPT)])


@jax.jit
def _sc_aggregate(emb2, cols1, rows4, vals1):
    mesh = plsc.VectorSubcoreMesh(core_axis_name="c", subcore_axis_name="s")
    return pl.kernel(
        _sc_aggregate_body,
        out_type=jax.ShapeDtypeStruct((NC, NPAD, H), jnp.float32),
        mesh=mesh,
        scratch_types=[
            pltpu.VMEM((2, 1, CHW), jnp.int32),
            pltpu.VMEM((NI, 1, B), jnp.int32),
            pltpu.VMEM((2, 1, CHW), jnp.float32),
            pltpu.VMEM((NBUF, B, H), jnp.float32),
            pltpu.VMEM_SHARED((NPAD, H), jnp.float32),
        ] + [pltpu.SemaphoreType.DMA] * (NI + 2 * NBUF + 1),
    )(emb2, cols1, rows4, vals1)


def _tc_mm1_body(emb_r, w1_r, out_r):
    out_r[...] = jnp.dot(emb_r[...], w1_r[...],
                         preferred_element_type=jnp.float32)


@jax.jit
def _tc_mm1(emb, w1t):
    blk = 1000
    return pl.pallas_call(
        _tc_mm1_body,
        grid=(N // blk,),
        in_specs=[
            pl.BlockSpec((blk, D), lambda i: (i, 0)),
            pl.BlockSpec((D, D), lambda i: (0, 0)),
        ],
        out_specs=pl.BlockSpec((blk, D), lambda i: (i, 0)),
        out_shape=jax.ShapeDtypeStruct((N, D), jnp.float32),
    )(emb, w1t)


def _tc_mm2_body(a_r, n0_r, n1_r, w2a_r, w2b_r, out_r):
    x = a_r[...]
    x += jnp.dot(n0_r[0], w2a_r[...], preferred_element_type=jnp.float32)
    x += jnp.dot(n1_r[0], w2b_r[...], preferred_element_type=jnp.float32)
    out_r[...] = jnp.where(x >= 0, x, 0.2 * x)


@jax.jit
def _tc_mm2(a, nb, w2ta, w2tb):
    blk = 1000
    return pl.pallas_call(
        _tc_mm2_body,
        grid=(N // blk,),
        in_specs=[
            pl.BlockSpec((blk, D), lambda i: (i, 0)),
            pl.BlockSpec((1, blk, H), lambda i: (0, i, 0)),
            pl.BlockSpec((1, blk, H), lambda i: (1, i, 0)),
            pl.BlockSpec((H, D), lambda i: (0, 0)),
            pl.BlockSpec((H, D), lambda i: (0, 0)),
        ],
        out_specs=pl.BlockSpec((blk, D), lambda i: (i, 0)),
        out_shape=jax.ShapeDtypeStruct((N, D), jnp.float32),
    )(a, nb, nb, w2ta, w2tb)


def kernel(emb, adj_indices, adj_values, W1, W2):
    rows = adj_indices[0]
    cols = adj_indices[1]
    pad = EPAD - E
    rows_p = jnp.concatenate([rows, jnp.zeros((pad,), jnp.int32)])
    cols_p = jnp.concatenate([cols, jnp.zeros((pad,), jnp.int32)])
    vals_p = jnp.concatenate([adj_values, jnp.zeros((pad,), jnp.float32)])

    emb2 = emb.reshape(N * NC, H)
    colsx = cols_p * 2
    cols4 = jnp.stack([colsx, colsx + 1]).reshape(NC, NS, NB, B)
    cols1 = jnp.pad(cols4, ((0, 0), (0, 0), (0, 0), (0, BP - B))).reshape(-1)
    rows4 = rows_p.reshape(NS, NB, 1, B)
    vals1 = jnp.pad(vals_p.reshape(NS, NB, B),
                    ((0, 0), (0, 0), (0, BP - B))).reshape(-1)

    nb = _sc_aggregate(emb2, cols1, rows4, vals1)
    a = _tc_mm1(emb, W1.T)
    return _tc_mm2(a, nb, W2[:, :H].T, W2[:, H:].T)
